# Initial kernel scaffold; baseline (speedup 1.0000x reference)
import functools

import jax
import jax.numpy as jnp
from jax import lax
from jax.experimental import pallas as pl
from jax.experimental.pallas import tpu as pltpu
from jax.experimental.pallas import tpu_sc as plsc

N_NODES = 10000
N_EDGES = 320000
D = 128

NC = 2
NS = 16
L = 16
NW = NC * NS

EDGES_PER_WORKER = N_EDGES // NW
CHUNK = 128
FULL_CHUNKS = EDGES_PER_WORKER // CHUNK
TAIL = EDGES_PER_WORKER - FULL_CHUNKS * CHUNK
ROWS_PER_TILE = N_NODES // NS
ZERO_ROWS = 125


def _sc_aggregate(x, rows, cols, vals):
    mesh = plsc.VectorSubcoreMesh(
        core_axis_name="c", subcore_axis_name="s",
        num_cores=NC, num_subcores=NS)

    @functools.partial(
        pl.kernel,
        out_type=jax.ShapeDtypeStruct((NC, N_NODES, D), jnp.float32),
        mesh=mesh,
        scratch_types=[
            pltpu.VMEM_SHARED((N_NODES, D), jnp.float32),
            pltpu.VMEM((CHUNK, D), jnp.float32),
            pltpu.VMEM((CHUNK,), jnp.int32),
            pltpu.VMEM((CHUNK,), jnp.int32),
            pltpu.VMEM((CHUNK,), jnp.float32),
            pltpu.VMEM((TAIL, D), jnp.float32),
            pltpu.VMEM((TAIL,), jnp.int32),
            pltpu.VMEM((TAIL,), jnp.int32),
            pltpu.VMEM((TAIL,), jnp.float32),
            pltpu.SemaphoreType.DMA,
        ],
    )
    def agg(x_hbm, rows_hbm, cols_hbm, vals_hbm, out_hbm,
            h_sh, gbuf, colb, rowb, valb, gbuf_t, colb_t, rowb_t, valb_t,
            sem):
        c = lax.axis_index("c")
        s = lax.axis_index("s")
        wid = c * NS + s

        def zero_row(r, _):
            for j in range(D // L):
                gbuf[r, pl.ds(j * L, L)] = jnp.zeros((L,), jnp.float32)
            return 0
        lax.fori_loop(0, ZERO_ROWS, zero_row, 0)
        for k in range(ROWS_PER_TILE // ZERO_ROWS):
            pltpu.sync_copy(
                gbuf.at[pl.ds(0, ZERO_ROWS)],
                h_sh.at[pl.ds(s * ROWS_PER_TILE + k * ZERO_ROWS, ZERO_ROWS)])
        plsc.subcore_barrier()

        def scale_rows(gb, vb, n):
            def body(r, _):
                vsplat = plsc.load_gather(
                    vb, [jnp.full((L,), r, jnp.int32)])
                for j in range(D // L):
                    gb[r, pl.ds(j * L, L)] = gb[r, pl.ds(j * L, L)] * vsplat
                return 0
            lax.fori_loop(0, n, body, 0)

        def do_chunk(k, _):
            base = wid * EDGES_PER_WORKER + k * CHUNK
            pltpu.sync_copy(cols_hbm.at[pl.ds(base, CHUNK)], colb)
            pltpu.sync_copy(rows_hbm.at[pl.ds(base, CHUNK)], rowb)
            pltpu.sync_copy(vals_hbm.at[pl.ds(base, CHUNK)], valb)
            pltpu.async_copy(x_hbm.at[colb], gbuf, sem).wait()
            scale_rows(gbuf, valb, CHUNK)
            pltpu.sync_copy(gbuf, h_sh.at[rowb], add=True)
            return 0
        lax.fori_loop(0, FULL_CHUNKS, do_chunk, 0)

        tbase = wid * EDGES_PER_WORKER + FULL_CHUNKS * CHUNK
        pltpu.sync_copy(cols_hbm.at[pl.ds(tbase, TAIL)], colb_t)
        pltpu.sync_copy(rows_hbm.at[pl.ds(tbase, TAIL)], rowb_t)
        pltpu.sync_copy(vals_hbm.at[pl.ds(tbase, TAIL)], valb_t)
        pltpu.async_copy(x_hbm.at[colb_t], gbuf_t, sem).wait()
        scale_rows(gbuf_t, valb_t, TAIL)
        pltpu.sync_copy(gbuf_t, h_sh.at[rowb_t], add=True)

        plsc.subcore_barrier()

        pltpu.sync_copy(
            h_sh.at[pl.ds(s * ROWS_PER_TILE, ROWS_PER_TILE)],
            out_hbm.at[c, pl.ds(s * ROWS_PER_TILE, ROWS_PER_TILE)])

    return agg(x, rows, cols, vals)


def _tc_matmul_relu(h_partial, W):
    BLOCK = 1000

    def mm(h_ref, w_ref, o_ref):
        hp = h_ref[...]
        y = hp[0] + hp[1]
        o_ref[...] = jnp.maximum(
            jnp.dot(y, w_ref[...], preferred_element_type=jnp.float32), 0.0)

    return pl.pallas_call(
        mm,
        grid=(N_NODES // BLOCK,),
        in_specs=[
            pl.BlockSpec((NC, BLOCK, D), lambda i: (0, i, 0)),
            pl.BlockSpec((D, D), lambda i: (0, 0)),
        ],
        out_specs=pl.BlockSpec((BLOCK, D), lambda i: (i, 0)),
        out_shape=jax.ShapeDtypeStruct((N_NODES, D), jnp.float32),
    )(h_partial, W)


def kernel(input, adj_indices, adj_values, W):
    rows = adj_indices[0]
    cols = adj_indices[1]
    h_partial = _sc_aggregate(input, rows, cols, adj_values)
    return _tc_matmul_relu(h_partial, W)


# speedup vs baseline: 5.5497x; 5.5497x over previous
import functools

import jax
import jax.numpy as jnp
from jax import lax
from jax.experimental import pallas as pl
from jax.experimental.pallas import tpu as pltpu
from jax.experimental.pallas import tpu_sc as plsc

N_NODES = 10000
N_EDGES = 320000
D = 128

NC = 2
NS = 16
L = 16
NW = NC * NS

EDGES_PER_WORKER = N_EDGES // NW
CHUNK = 128
FULL_CHUNKS = EDGES_PER_WORKER // CHUNK
TAIL = EDGES_PER_WORKER - FULL_CHUNKS * CHUNK
HCHUNK = 128
N_HCHUNKS = N_NODES // HCHUNK
HROUNDS = (N_HCHUNKS + NS - 1) // NS
HTAIL = N_NODES - N_HCHUNKS * HCHUNK


def _sc_aggregate(x, rows, cols, vals):
    mesh = plsc.VectorSubcoreMesh(
        core_axis_name="c", subcore_axis_name="s",
        num_cores=NC, num_subcores=NS)

    @functools.partial(
        pl.kernel,
        out_type=jax.ShapeDtypeStruct((NC, N_NODES, D), jnp.float32),
        mesh=mesh,
        scratch_types=[
            pltpu.VMEM_SHARED((N_NODES, D), jnp.float32),
            pltpu.VMEM((CHUNK, D), jnp.float32),
            pltpu.VMEM((CHUNK,), jnp.int32),
            pltpu.VMEM((CHUNK,), jnp.int32),
            pltpu.VMEM((CHUNK,), jnp.float32),
            pltpu.VMEM((TAIL, D), jnp.float32),
            pltpu.VMEM((TAIL,), jnp.int32),
            pltpu.VMEM((TAIL,), jnp.int32),
            pltpu.VMEM((TAIL,), jnp.float32),
            pltpu.SemaphoreType.DMA,
        ],
    )
    def agg(x_hbm, rows_hbm, cols_hbm, vals_hbm, out_hbm,
            h_sh, gbuf, colb, rowb, valb, gbuf_t, colb_t, rowb_t, valb_t,
            sem):
        c = lax.axis_index("c")
        s = lax.axis_index("s")
        wid = c * NS + s

        def zero_row(r, _):
            for j in range(D // L):
                gbuf[r, pl.ds(j * L, L)] = jnp.zeros((L,), jnp.float32)
            return 0
        lax.fori_loop(0, HCHUNK, zero_row, 0)
        for k in range(HROUNDS):
            cid = s + NS * k

            @pl.when(cid < N_HCHUNKS)
            def _():
                pltpu.sync_copy(gbuf, h_sh.at[pl.ds(cid * HCHUNK, HCHUNK)])

        @pl.when(s == NS - 1)
        def _():
            pltpu.sync_copy(gbuf.at[pl.ds(0, HTAIL)],
                            h_sh.at[pl.ds(N_HCHUNKS * HCHUNK, HTAIL)])
        plsc.subcore_barrier()

        def scale_rows(gb, vb, n):
            def body(g, _):
                v16 = vb[pl.ds(g * L, L)]
                for i in range(L):
                    r = g * L + i
                    vs = v16[i]
                    for j in range(D // L):
                        gb[r, pl.ds(j * L, L)] = gb[r, pl.ds(j * L, L)] * vs
                return 0
            lax.fori_loop(0, n // L, body, 0)

        def do_chunk(k, _):
            base = wid * EDGES_PER_WORKER + k * CHUNK
            pltpu.sync_copy(cols_hbm.at[pl.ds(base, CHUNK)], colb)
            pltpu.sync_copy(rows_hbm.at[pl.ds(base, CHUNK)], rowb)
            pltpu.sync_copy(vals_hbm.at[pl.ds(base, CHUNK)], valb)
            pltpu.async_copy(x_hbm.at[colb], gbuf, sem).wait()
            scale_rows(gbuf, valb, CHUNK)
            pltpu.sync_copy(gbuf, h_sh.at[rowb], add=True)
            return 0
        lax.fori_loop(0, FULL_CHUNKS, do_chunk, 0)

        tbase = wid * EDGES_PER_WORKER + FULL_CHUNKS * CHUNK
        pltpu.sync_copy(cols_hbm.at[pl.ds(tbase, TAIL)], colb_t)
        pltpu.sync_copy(rows_hbm.at[pl.ds(tbase, TAIL)], rowb_t)
        pltpu.sync_copy(vals_hbm.at[pl.ds(tbase, TAIL)], valb_t)
        pltpu.async_copy(x_hbm.at[colb_t], gbuf_t, sem).wait()
        scale_rows(gbuf_t, valb_t, TAIL)
        pltpu.sync_copy(gbuf_t, h_sh.at[rowb_t], add=True)

        plsc.subcore_barrier()

        for k in range(HROUNDS):
            cid = s + NS * k

            @pl.when(cid < N_HCHUNKS)
            def _():
                pltpu.sync_copy(h_sh.at[pl.ds(cid * HCHUNK, HCHUNK)],
                                out_hbm.at[c, pl.ds(cid * HCHUNK, HCHUNK)])

        @pl.when(s == NS - 1)
        def _():
            pltpu.sync_copy(h_sh.at[pl.ds(N_HCHUNKS * HCHUNK, HTAIL)],
                            out_hbm.at[c, pl.ds(N_HCHUNKS * HCHUNK, HTAIL)])

    return agg(x, rows, cols, vals)


def _tc_matmul_relu(h_partial, W):
    BLOCK = 1000

    def mm(h_ref, w_ref, o_ref):
        hp = h_ref[...]
        y = hp[0] + hp[1]
        o_ref[...] = jnp.maximum(
            jnp.dot(y, w_ref[...], preferred_element_type=jnp.float32), 0.0)

    return pl.pallas_call(
        mm,
        grid=(N_NODES // BLOCK,),
        in_specs=[
            pl.BlockSpec((NC, BLOCK, D), lambda i: (0, i, 0)),
            pl.BlockSpec((D, D), lambda i: (0, 0)),
        ],
        out_specs=pl.BlockSpec((BLOCK, D), lambda i: (i, 0)),
        out_shape=jax.ShapeDtypeStruct((N_NODES, D), jnp.float32),
    )(h_partial, W)


def kernel(input, adj_indices, adj_values, W):
    rows = adj_indices[0]
    cols = adj_indices[1]
    h_partial = _sc_aggregate(input, rows, cols, adj_values)
    return _tc_matmul_relu(h_partial, W)
